# Initial kernel scaffold; baseline (speedup 1.0000x reference)
import jax
import jax.numpy as jnp
from jax.experimental import pallas as pl


def kernel(x, W_g, W_s):
    raise NotImplementedError("write your pallas kernel here")


# speedup vs baseline: 3.1515x; 3.1515x over previous
import functools

import jax
import jax.numpy as jnp
from jax import lax
from jax.experimental import pallas as pl
from jax.experimental.pallas import tpu as pltpu
from jax.experimental.pallas import tpu_sc as plsc

NC = 2
NSC = 16
NW = NC * NSC

CTX = 50
NB = 8
IDX_DMA = 80


def _sc_partials(B, D, S, V):
    per_w = B // NW
    nch = per_w // NB
    rows_per_chunk = NB * CTX
    ndma = rows_per_chunk // IDX_DMA
    SD = S * D

    mesh = plsc.VectorSubcoreMesh(core_axis_name="c", subcore_axis_name="s")

    @functools.partial(
        pl.kernel,
        out_type=jax.ShapeDtypeStruct((NW, S, 16), jnp.float32),
        mesh=mesh,
        compiler_params=pltpu.CompilerParams(use_tc_tiling_on_sc=False),
        scratch_types=[
            pltpu.VMEM((per_w * CTX,), jnp.int32),
            pltpu.VMEM((per_w,), jnp.int32),
            pltpu.VMEM((rows_per_chunk, D), jnp.float32),
            pltpu.VMEM((NB, SD), jnp.float32),
            pltpu.VMEM((S, 16), jnp.float32),
            pltpu.SemaphoreType.DMA,
        ],
    )
    def k(ctx_hbm, wid_hbm, wg_hbm, ws_hbm, out_hbm,
          idx_v, widx_v, rows_v, sense_v, acc_v, sem):
        w = lax.axis_index("s") * NC + lax.axis_index("c")
        base = w * per_w
        pltpu.sync_copy(ctx_hbm.at[pl.ds(base * CTX, per_w * CTX)], idx_v)
        pltpu.sync_copy(wid_hbm.at[pl.ds(base, per_w)], widx_v)

        zero = jnp.zeros((16,), jnp.float32)
        nd = D // 16

        def chunk_body(ci, saccs):
            saccs = list(saccs)
            copies = []
            for j in range(ndma):
                copies.append(pltpu.async_copy(
                    wg_hbm.at[idx_v.at[pl.ds(ci * rows_per_chunk + j * IDX_DMA,
                                             IDX_DMA)]],
                    rows_v.at[pl.ds(j * IDX_DMA, IDX_DMA)], sem))
            copies.append(pltpu.async_copy(
                ws_hbm.at[widx_v.at[pl.ds(ci * NB, NB)]], sense_v, sem))
            for c in copies:
                c.wait()
            for b in range(NB):
                def rbody(r, accs, b=b):
                    row = b * CTX + r
                    return tuple(accs[d] + rows_v[row, pl.ds(d * 16, 16)]
                                 for d in range(nd))
                accs = lax.fori_loop(0, CTX, rbody, (zero,) * nd)
                for si in range(S):
                    for d in range(nd):
                        saccs[si] = saccs[si] + (
                            sense_v[b, pl.ds(si * D + d * 16, 16)] * accs[d])
            return tuple(saccs)

        saccs = lax.fori_loop(0, nch, chunk_body, (zero,) * S)
        for si in range(S):
            acc_v[si] = saccs[si]
        pltpu.sync_copy(acc_v, out_hbm.at[w])

    return k


def _tc_finish(p_ref, o_ref):
    o_ref[...] = jax.nn.sigmoid(
        jnp.sum(p_ref[...], axis=(0, 2)).reshape(1, -1))


def kernel(x, W_g, W_s):
    B, _ = x.shape
    V, D = W_g.shape
    S = W_s.shape[1]
    ctx = x[:, 2:].reshape(-1)
    wid = x[:, 0]
    ws2 = W_s.reshape(V, S * D)
    partials = _sc_partials(B, D, S, V)(ctx, wid, W_g, ws2)
    out = pl.pallas_call(
        _tc_finish,
        out_shape=jax.ShapeDtypeStruct((1, S), jnp.float32),
    )(partials)
    return out[0]
